# baseline (speedup 1.0000x reference)
import functools

import jax
import jax.numpy as jnp
from jax import lax
from jax.experimental import pallas as pl
from jax.experimental.pallas import tpu as pltpu
from jax.experimental.pallas import tpu_sc as plsc

EMB = 32
N = 4096 * 200
T4 = 250000
NW = 32
PER_W = N // NW
NBUF = 2
CHUNK = 320
NCHUNK = PER_W // CHUNK
NOUTER = NCHUNK // NBUF
GROUPS = CHUNK // 16


def _make_gather():
    mesh = plsc.VectorSubcoreMesh(core_axis_name="c", subcore_axis_name="s")

    scratch = (
        [pltpu.VMEM((CHUNK,), jnp.int32) for _ in range(NBUF)]
        + [pltpu.VMEM((CHUNK,), jnp.int32) for _ in range(NBUF)]
        + [pltpu.VMEM((CHUNK, 128), jnp.float32) for _ in range(NBUF)]
        + [pltpu.VMEM((CHUNK * EMB,), jnp.float32) for _ in range(NBUF)]
        + [pltpu.SemaphoreType.DMA for _ in range(2 * NBUF)]
    )

    @functools.partial(
        pl.kernel,
        mesh=mesh,
        out_type=jax.ShapeDtypeStruct((N * EMB,), jnp.float32),
        scratch_types=scratch,
        compiler_params=pltpu.CompilerParams(needs_layout_passes=False),
    )
    def gather_kernel(ids_hbm, table_hbm, out_hbm, *scratch_refs):
        idx_v = scratch_refs[:NBUF]
        line_v = scratch_refs[NBUF:2 * NBUF]
        lines_v = scratch_refs[2 * NBUF:3 * NBUF]
        out_v = scratch_refs[3 * NBUF:4 * NBUF]
        gsem = scratch_refs[4 * NBUF:5 * NBUF]
        wsem = scratch_refs[5 * NBUF:6 * NBUF]

        wid = lax.axis_index("s") * 2 + lax.axis_index("c")
        base = wid * PER_W
        iota = lax.iota(jnp.int32, 16)

        def stage_and_fire(b, i):
            pltpu.sync_copy(ids_hbm.at[pl.ds(base + i * CHUNK, CHUNK)],
                            idx_v[b])
            for g in range(GROUPS):
                iv = idx_v[b][pl.ds(g * 16, 16)]
                line_v[b][pl.ds(g * 16, 16)] = lax.shift_right_logical(iv, 2)
            pltpu.async_copy(table_hbm.at[line_v[b]], lines_v[b], gsem[b])

        def gather_wait(b):
            pltpu.make_async_copy(
                table_hbm.at[line_v[b]], lines_v[b], gsem[b]).wait()

        def extract(b):
            def g_body(g, carry):
                iv = idx_v[b][pl.ds(g * 16, 16)]
                sub32 = lax.shift_left(iv & 3, 5)
                rows = iota + g * 16
                dst0 = iota * EMB + g * (16 * EMB)
                for c in range(EMB):
                    val = plsc.load_gather(lines_v[b], [rows, sub32 + c])
                    plsc.store_scatter(out_v[b], [dst0 + c], val)
                return carry

            lax.fori_loop(0, GROUPS, g_body, 0)

        def write_start(b, i):
            pltpu.async_copy(
                out_v[b],
                out_hbm.at[pl.ds((base + i * CHUNK) * EMB, CHUNK * EMB)],
                wsem[b])

        def write_wait(b, i):
            pltpu.make_async_copy(
                out_v[b],
                out_hbm.at[pl.ds((base + i * CHUNK) * EMB, CHUNK * EMB)],
                wsem[b]).wait()

        for b in range(NBUF):
            stage_and_fire(b, b)
        for b in range(NBUF):
            gather_wait(b)
            extract(b)
            write_start(b, b)
            stage_and_fire(b, b + NBUF)

        def outer_body(g, carry):
            for b in range(NBUF):
                i = g * NBUF + b
                gather_wait(b)
                write_wait(b, i - NBUF)
                extract(b)
                write_start(b, i)
                stage_and_fire(b, i + NBUF)
            return carry

        lax.fori_loop(1, NOUTER - 1, outer_body, 0)

        last = (NOUTER - 1) * NBUF
        for b in range(NBUF):
            gather_wait(b)
            write_wait(b, last + b - NBUF)
            extract(b)
            write_start(b, last + b)
        for b in range(NBUF):
            write_wait(b, last + b)

    return gather_kernel


_gather = _make_gather()


def kernel(word_ids, table):
    flat = word_ids.reshape(-1)
    table4 = table.reshape(T4, 4 * EMB)
    out = _gather(flat, table4)
    return out.reshape(word_ids.shape + (EMB,))


# speedup vs baseline: 2.0330x; 2.0330x over previous
import functools

import jax
import jax.numpy as jnp
from jax import lax
from jax.experimental import pallas as pl
from jax.experimental.pallas import tpu as pltpu
from jax.experimental.pallas import tpu_sc as plsc

EMB = 32
B = 4096
L = 200
N = B * L
NW = 32
PER_W = N // NW
NBUF = 4
CHUNK = 800
NCHUNK = PER_W // CHUNK
NOUTER = NCHUNK // NBUF


def _make_gather():
    mesh = plsc.VectorSubcoreMesh(core_axis_name="c", subcore_axis_name="s")

    scratch = (
        [pltpu.VMEM((CHUNK,), jnp.int32) for _ in range(NBUF)]
        + [pltpu.VMEM((CHUNK, EMB), jnp.float32) for _ in range(NBUF)]
        + [pltpu.SemaphoreType.DMA for _ in range(3 * NBUF)]
    )

    @functools.partial(
        pl.kernel,
        mesh=mesh,
        out_type=jax.ShapeDtypeStruct((N, EMB), jnp.float32),
        scratch_types=scratch,
        compiler_params=pltpu.CompilerParams(use_tc_tiling_on_sc=False),
    )
    def gather_kernel(ids_hbm, table_hbm, out_hbm, *scratch_refs):
        idx_v = scratch_refs[:NBUF]
        rows_v = scratch_refs[NBUF:2 * NBUF]
        isem = scratch_refs[2 * NBUF:3 * NBUF]
        gsem = scratch_refs[3 * NBUF:4 * NBUF]
        wsem = scratch_refs[4 * NBUF:5 * NBUF]

        wid = lax.axis_index("s") * 2 + lax.axis_index("c")
        base = wid * PER_W

        def idx_start(b, i):
            pltpu.async_copy(
                ids_hbm.at[pl.ds(base + i * CHUNK, CHUNK)], idx_v[b], isem[b])

        def idx_wait(b, i):
            pltpu.make_async_copy(
                ids_hbm.at[pl.ds(base + i * CHUNK, CHUNK)], idx_v[b],
                isem[b]).wait()

        def gather_start(b):
            pltpu.async_copy(table_hbm.at[idx_v[b]], rows_v[b], gsem[b])

        def gather_wait(b):
            pltpu.make_async_copy(
                table_hbm.at[idx_v[b]], rows_v[b], gsem[b]).wait()

        def write_start(b, i):
            pltpu.async_copy(
                rows_v[b], out_hbm.at[pl.ds(base + i * CHUNK, CHUNK)], wsem[b])

        def write_wait(b, i):
            pltpu.make_async_copy(
                rows_v[b], out_hbm.at[pl.ds(base + i * CHUNK, CHUNK)],
                wsem[b]).wait()

        for b in range(NBUF):
            idx_start(b, b)
        for b in range(NBUF):
            idx_wait(b, b)
            gather_start(b)

        def outer_body(g, carry):
            for b in range(NBUF):
                i = g * NBUF + b
                j = i + NBUF
                gather_wait(b)
                idx_start(b, j)
                write_start(b, i)
                write_wait(b, i)
                idx_wait(b, j)
                gather_start(b)
            return carry

        lax.fori_loop(0, NOUTER - 1, outer_body, 0)

        last = (NOUTER - 1) * NBUF
        for b in range(NBUF):
            gather_wait(b)
            write_start(b, last + b)
        for b in range(NBUF):
            write_wait(b, last + b)

    return gather_kernel


_gather = _make_gather()


def kernel(word_ids, table):
    flat = word_ids.T.reshape(-1)
    out = _gather(flat, table)
    return out.reshape(L, B, EMB).transpose(1, 0, 2)
